# Initial kernel scaffold; baseline (speedup 1.0000x reference)
import jax
import jax.numpy as jnp
from jax.experimental import pallas as pl


def kernel(h, edge_index, W1, b1, W2, b2, W3, b3, Wc, bc):
    raise NotImplementedError("write your pallas kernel here")


# speedup vs baseline: 14.3189x; 14.3189x over previous
import functools

import jax
import jax.numpy as jnp
from jax import lax
from jax.experimental import pallas as pl
from jax.experimental.pallas import tpu as pltpu
from jax.experimental.pallas import tpu_sc as plsc

N = 100000
E = 1600000
F_IN = 34

NC = 2
NS = 16
NW = NC * NS
CHUNK = 128
EPW = 50048
EPAD = EPW * NW
NCHUNKS = EPW // CHUNK
NPAD = 102400
RPS = NPAD // NS

_MESH = plsc.VectorSubcoreMesh(
    core_axis_name="c", subcore_axis_name="s", num_cores=NC, num_subcores=NS
)
_SC_PARAMS = pltpu.CompilerParams(use_tc_tiling_on_sc=False)


@functools.partial(
    pl.kernel,
    out_type=jax.ShapeDtypeStruct((NC, NPAD), jnp.float32),
    mesh=_MESH,
    compiler_params=_SC_PARAMS,
    scratch_types=[
        pltpu.VMEM((CHUNK,), jnp.int32),
        pltpu.VMEM((CHUNK,), jnp.float32),
        pltpu.VMEM((RPS,), jnp.float32),
        pltpu.VMEM_SHARED((NPAD,), jnp.float32),
    ],
)
def _deg_kernel(dst_hbm, out_hbm, dst_v, ones_v, zb_v, acc_sh):
    c = lax.axis_index("c")
    s = lax.axis_index("s")
    wid = s * NC + c

    def _fill_ones(i, carry):
        ones_v[pl.ds(i * 16, 16)] = jnp.full((16,), 1.0, jnp.float32)
        return carry

    lax.fori_loop(0, CHUNK // 16, _fill_ones, 0)

    def _fill_zero(i, carry):
        zb_v[pl.ds(i * 16, 16)] = jnp.zeros((16,), jnp.float32)
        return carry

    lax.fori_loop(0, RPS // 16, _fill_zero, 0)

    pltpu.sync_copy(zb_v, acc_sh.at[pl.ds(s * RPS, RPS)])
    plsc.subcore_barrier()

    base = wid * EPW

    def _body(g, carry):
        pltpu.sync_copy(dst_hbm.at[pl.ds(base + g * CHUNK, CHUNK)], dst_v)
        pltpu.sync_copy(ones_v, acc_sh.at[dst_v], add=True)
        return carry

    lax.fori_loop(0, NCHUNKS, _body, 0)

    plsc.subcore_barrier()
    pltpu.sync_copy(acc_sh.at[pl.ds(s * RPS, RPS)], out_hbm.at[c, pl.ds(s * RPS, RPS)])


def _make_edge_kernel(f):
    @functools.partial(
        pl.kernel,
        out_type=jax.ShapeDtypeStruct((NC, NPAD, f), jnp.float32),
        mesh=_MESH,
        compiler_params=_SC_PARAMS,
        scratch_types=[
            pltpu.VMEM((CHUNK,), jnp.int32),
            pltpu.VMEM((CHUNK,), jnp.int32),
            pltpu.VMEM((CHUNK, f), jnp.float32),
            pltpu.VMEM_SHARED((NPAD, f), jnp.float32),
            pltpu.SemaphoreType.DMA,
        ],
    )
    def _edge_kernel(y_hbm, src_hbm, dst_hbm, out_hbm, src_v, dst_v, rows_v, acc_sh, sem):
        c = lax.axis_index("c")
        s = lax.axis_index("s")
        wid = s * NC + c
        r0 = s * RPS

        pltpu.sync_copy(y_hbm.at[pl.ds(r0, RPS)], acc_sh.at[pl.ds(r0, RPS)])
        plsc.subcore_barrier()

        base = wid * EPW

        def _body(g, carry):
            e0 = base + g * CHUNK
            pltpu.sync_copy(src_hbm.at[pl.ds(e0, CHUNK)], src_v)
            pltpu.sync_copy(dst_hbm.at[pl.ds(e0, CHUNK)], dst_v)
            pltpu.async_copy(y_hbm.at[src_v], rows_v, sem).wait()
            pltpu.sync_copy(rows_v, acc_sh.at[dst_v], add=True)
            return carry

        lax.fori_loop(0, NCHUNKS, _body, 0)

        plsc.subcore_barrier()
        pltpu.sync_copy(
            acc_sh.at[pl.ds(r0, RPS)], out_hbm.at[c, pl.ds(r0, RPS)]
        )

    return _edge_kernel


_edge16 = _make_edge_kernel(16)


BR = 2048
GRID = NPAD // BR


def _row_spec(f):
    return pl.BlockSpec((BR, f), lambda i: (i, 0))


def _full_spec(r, f):
    return pl.BlockSpec((r, f), lambda i: (0, 0))


def _t1_body(d0_ref, d1_ref, h_ref, w_ref, dinv_ref, y_ref):
    dinv = lax.rsqrt(d0_ref[...] + d1_ref[...] + 1.0)
    dinv_ref[...] = dinv
    y_ref[...] = (
        jnp.dot(h_ref[...], w_ref[...], preferred_element_type=jnp.float32) * dinv
    )


def _t1(d0, d1, hpad, W1):
    return pl.pallas_call(
        _t1_body,
        grid=(GRID,),
        in_specs=[_row_spec(1), _row_spec(1), _row_spec(F_IN), _full_spec(F_IN, 16)],
        out_specs=[_row_spec(1), _row_spec(16)],
        out_shape=[
            jax.ShapeDtypeStruct((NPAD, 1), jnp.float32),
            jax.ShapeDtypeStruct((NPAD, 16), jnp.float32),
        ],
    )(d0, d1, hpad, W1)


def _make_tmid_body(fi, fo):
    def _tmid_body(y_ref, p0_ref, p1_ref, b_ref, dinv_ref, w_ref, ynext_ref):
        agg = p0_ref[:, :fi] + p1_ref[:, :fi] - y_ref[:, :fi]
        hn = jnp.tanh(dinv_ref[...] * agg + b_ref[...])
        yn = jnp.dot(hn, w_ref[...], preferred_element_type=jnp.float32) * dinv_ref[...]
        ynext_ref[...] = jnp.concatenate(
            [yn, jnp.zeros((yn.shape[0], 16 - fo), jnp.float32)], axis=1
        )

    return _tmid_body


def _tmid(y, p0, p1, b, dinv, W, fi, fo):
    return pl.pallas_call(
        _make_tmid_body(fi, fo),
        grid=(GRID,),
        in_specs=[
            _row_spec(16),
            _row_spec(16),
            _row_spec(16),
            _full_spec(1, fi),
            _row_spec(1),
            _full_spec(fi, fo),
        ],
        out_specs=_row_spec(16),
        out_shape=jax.ShapeDtypeStruct((NPAD, 16), jnp.float32),
    )(y, p0, p1, b, dinv, W)


def _t4_body(y_ref, p0_ref, p1_ref, b_ref, dinv_ref, wc_ref, bc_ref, out_ref, h3_ref):
    h3 = jnp.tanh(
        dinv_ref[...] * (p0_ref[:, :2] + p1_ref[:, :2] - y_ref[:, :2]) + b_ref[...]
    )
    h3_ref[...] = h3
    out_ref[...] = (
        jnp.dot(h3, wc_ref[...], preferred_element_type=jnp.float32) + bc_ref[...]
    )


def _t4(y, p0, p1, b, dinv, Wc, bc):
    return pl.pallas_call(
        _t4_body,
        grid=(GRID,),
        in_specs=[
            _row_spec(16),
            _row_spec(16),
            _row_spec(16),
            _full_spec(1, 2),
            _row_spec(1),
            _full_spec(2, 4),
            _full_spec(1, 4),
        ],
        out_specs=[_row_spec(4), _row_spec(2)],
        out_shape=[
            jax.ShapeDtypeStruct((NPAD, 4), jnp.float32),
            jax.ShapeDtypeStruct((NPAD, 2), jnp.float32),
        ],
    )(y, p0, p1, b, dinv, Wc, bc)


def kernel(h, edge_index, W1, b1, W2, b2, W3, b3, Wc, bc):
    src = edge_index[0]
    dst = edge_index[1]
    pad_idx = jnp.full((EPAD - E,), N, dtype=jnp.int32)
    src_p = jnp.concatenate([src, pad_idx])
    dst_p = jnp.concatenate([dst, pad_idx])
    hpad = jnp.concatenate([h, jnp.zeros((NPAD - N, F_IN), h.dtype)])

    deg = _deg_kernel(dst_p)
    d0 = deg[0][:, None]
    d1 = deg[1][:, None]

    dinv, y1 = _t1(d0, d1, hpad, W1)
    p1 = _edge16(y1, src_p, dst_p)
    y2 = _tmid(y1, p1[0], p1[1], b1.reshape(1, -1), dinv, W2, 16, 4)
    p2 = _edge16(y2, src_p, dst_p)
    y3 = _tmid(y2, p2[0], p2[1], b2.reshape(1, -1), dinv, W3, 4, 2)
    p3 = _edge16(y3, src_p, dst_p)
    out, h3 = _t4(y3, p3[0], p3[1], b3.reshape(1, -1), dinv, Wc, bc.reshape(1, -1))
    return (out[:N], h3[:N])


# speedup vs baseline: 27.6242x; 1.9292x over previous
import functools

import jax
import jax.numpy as jnp
from jax import lax
from jax.experimental import pallas as pl
from jax.experimental.pallas import tpu as pltpu
from jax.experimental.pallas import tpu_sc as plsc

N = 100000
E = 1600000
F_IN = 34

NC = 2
NS = 16
NW = NC * NS
CHUNK = 128
NB = 8
EPW = 50176
EPAD = EPW * NW
NCHUNKS = EPW // CHUNK
NSTEPS = NCHUNKS // NB
NPAD = 102400
RPS = NPAD // NS

_MESH = plsc.VectorSubcoreMesh(
    core_axis_name="c", subcore_axis_name="s", num_cores=NC, num_subcores=NS
)
_SC_PARAMS = pltpu.CompilerParams(use_tc_tiling_on_sc=False)


@functools.partial(
    pl.kernel,
    out_type=jax.ShapeDtypeStruct((NC, NPAD, 16), jnp.float32),
    mesh=_MESH,
    compiler_params=_SC_PARAMS,
    scratch_types=[
        pltpu.VMEM((NB, CHUNK), jnp.int32),
        pltpu.VMEM((CHUNK, 16), jnp.float32),
        pltpu.VMEM_SHARED((NPAD, 16), jnp.float32),
        [pltpu.SemaphoreType.DMA] * NB,
    ],
)
def _deg_kernel(dst_hbm, out_hbm, dst_b, ones_v, acc_sh, ssems):
    c = lax.axis_index("c")
    s = lax.axis_index("s")
    wid = s * NC + c
    r0 = s * RPS

    def _fill_ones(i, carry):
        ones_v[i, :] = jnp.full((16,), 1.0, jnp.float32)
        return carry

    lax.fori_loop(0, CHUNK, _fill_ones, 0)

    def _init(r, carry):
        pltpu.sync_copy(ones_v, acc_sh.at[pl.ds(r0 + r * CHUNK, CHUNK)])
        return carry

    lax.fori_loop(0, RPS // CHUNK, _init, 0)
    plsc.subcore_barrier()

    row0 = wid * NCHUNKS

    def _step(st, carry):
        pltpu.sync_copy(dst_hbm.at[pl.ds(row0 + st * NB, NB)], dst_b)
        descs = [
            pltpu.async_copy(ones_v, acc_sh.at[dst_b.at[b]], ssems[b], add=True)
            for b in range(NB)
        ]
        for d in descs:
            d.wait()
        return carry

    lax.fori_loop(0, NSTEPS, _step, 0)

    plsc.subcore_barrier()
    pltpu.sync_copy(acc_sh.at[pl.ds(r0, RPS)], out_hbm.at[c, pl.ds(r0, RPS)])


def _make_edge_kernel(f):
    @functools.partial(
        pl.kernel,
        out_type=jax.ShapeDtypeStruct((NC, NPAD, f), jnp.float32),
        mesh=_MESH,
        compiler_params=_SC_PARAMS,
        scratch_types=[
            pltpu.VMEM((NB, CHUNK), jnp.int32),
            pltpu.VMEM((NB, CHUNK), jnp.int32),
            pltpu.VMEM((NB, CHUNK, f), jnp.float32),
            pltpu.VMEM_SHARED((NPAD, f), jnp.float32),
            [pltpu.SemaphoreType.DMA] * NB,
            [pltpu.SemaphoreType.DMA] * NB,
        ],
    )
    def _edge_kernel(
        y_hbm, src_hbm, dst_hbm, out_hbm, src_b, dst_b, rows_b, acc_sh, gsems, ssems
    ):
        c = lax.axis_index("c")
        s = lax.axis_index("s")
        wid = s * NC + c
        r0 = s * RPS

        pltpu.sync_copy(y_hbm.at[pl.ds(r0, RPS)], acc_sh.at[pl.ds(r0, RPS)])
        plsc.subcore_barrier()

        row0 = wid * NCHUNKS

        def _step(st, carry):
            pltpu.sync_copy(src_hbm.at[pl.ds(row0 + st * NB, NB)], src_b)
            pltpu.sync_copy(dst_hbm.at[pl.ds(row0 + st * NB, NB)], dst_b)
            gds = [
                pltpu.async_copy(y_hbm.at[src_b.at[b]], rows_b.at[b], gsems[b])
                for b in range(NB)
            ]
            sds = []
            for b in range(NB):
                gds[b].wait()
                sds.append(
                    pltpu.async_copy(
                        rows_b.at[b], acc_sh.at[dst_b.at[b]], ssems[b], add=True
                    )
                )
            for d in sds:
                d.wait()
            return carry

        lax.fori_loop(0, NSTEPS, _step, 0)

        plsc.subcore_barrier()
        pltpu.sync_copy(
            acc_sh.at[pl.ds(r0, RPS)], out_hbm.at[c, pl.ds(r0, RPS)]
        )

    return _edge_kernel


_edge16 = _make_edge_kernel(16)


BR = 2048
GRID = NPAD // BR


def _row_spec(f):
    return pl.BlockSpec((BR, f), lambda i: (i, 0))


def _full_spec(r, f):
    return pl.BlockSpec((r, f), lambda i: (0, 0))


def _t1_body(d0_ref, d1_ref, h_ref, w_ref, dinv_ref, y_ref):
    dinv = lax.rsqrt(d0_ref[...] + d1_ref[...] - 1.0)
    dinv_ref[...] = dinv
    y_ref[...] = (
        jnp.dot(h_ref[...], w_ref[...], preferred_element_type=jnp.float32) * dinv
    )


def _t1(d0, d1, hpad, W1):
    return pl.pallas_call(
        _t1_body,
        grid=(GRID,),
        in_specs=[_row_spec(1), _row_spec(1), _row_spec(F_IN), _full_spec(F_IN, 16)],
        out_specs=[_row_spec(1), _row_spec(16)],
        out_shape=[
            jax.ShapeDtypeStruct((NPAD, 1), jnp.float32),
            jax.ShapeDtypeStruct((NPAD, 16), jnp.float32),
        ],
    )(d0, d1, hpad, W1)


def _make_tmid_body(fi, fo):
    def _tmid_body(y_ref, p0_ref, p1_ref, b_ref, dinv_ref, w_ref, ynext_ref):
        agg = p0_ref[:, :fi] + p1_ref[:, :fi] - y_ref[:, :fi]
        hn = jnp.tanh(dinv_ref[...] * agg + b_ref[...])
        yn = jnp.dot(hn, w_ref[...], preferred_element_type=jnp.float32) * dinv_ref[...]
        ynext_ref[...] = jnp.concatenate(
            [yn, jnp.zeros((yn.shape[0], 16 - fo), jnp.float32)], axis=1
        )

    return _tmid_body


def _tmid(y, p0, p1, b, dinv, W, fi, fo):
    return pl.pallas_call(
        _make_tmid_body(fi, fo),
        grid=(GRID,),
        in_specs=[
            _row_spec(16),
            _row_spec(16),
            _row_spec(16),
            _full_spec(1, fi),
            _row_spec(1),
            _full_spec(fi, fo),
        ],
        out_specs=_row_spec(16),
        out_shape=jax.ShapeDtypeStruct((NPAD, 16), jnp.float32),
    )(y, p0, p1, b, dinv, W)


def _t4_body(y_ref, p0_ref, p1_ref, b_ref, dinv_ref, wc_ref, bc_ref, out_ref, h3_ref):
    h3 = jnp.tanh(
        dinv_ref[...] * (p0_ref[:, :2] + p1_ref[:, :2] - y_ref[:, :2]) + b_ref[...]
    )
    h3_ref[...] = h3
    out_ref[...] = (
        jnp.dot(h3, wc_ref[...], preferred_element_type=jnp.float32) + bc_ref[...]
    )


def _t4(y, p0, p1, b, dinv, Wc, bc):
    return pl.pallas_call(
        _t4_body,
        grid=(GRID,),
        in_specs=[
            _row_spec(16),
            _row_spec(16),
            _row_spec(16),
            _full_spec(1, 2),
            _row_spec(1),
            _full_spec(2, 4),
            _full_spec(1, 4),
        ],
        out_specs=[_row_spec(4), _row_spec(2)],
        out_shape=[
            jax.ShapeDtypeStruct((NPAD, 4), jnp.float32),
            jax.ShapeDtypeStruct((NPAD, 2), jnp.float32),
        ],
    )(y, p0, p1, b, dinv, Wc, bc)


def kernel(h, edge_index, W1, b1, W2, b2, W3, b3, Wc, bc):
    src = edge_index[0]
    dst = edge_index[1]
    pad_idx = jnp.full((EPAD - E,), N, dtype=jnp.int32)
    src_p = jnp.concatenate([src, pad_idx]).reshape(EPAD // CHUNK, CHUNK)
    dst_p = jnp.concatenate([dst, pad_idx]).reshape(EPAD // CHUNK, CHUNK)
    hpad = jnp.concatenate([h, jnp.zeros((NPAD - N, F_IN), h.dtype)])

    deg = _deg_kernel(dst_p)
    d0 = deg[0, :, 0:1]
    d1 = deg[1, :, 0:1]

    dinv, y1 = _t1(d0, d1, hpad, W1)
    p1 = _edge16(y1, src_p, dst_p)
    y2 = _tmid(y1, p1[0], p1[1], b1.reshape(1, -1), dinv, W2, 16, 4)
    p2 = _edge16(y2, src_p, dst_p)
    y3 = _tmid(y2, p2[0], p2[1], b2.reshape(1, -1), dinv, W3, 4, 2)
    p3 = _edge16(y3, src_p, dst_p)
    out, h3 = _t4(y3, p3[0], p3[1], b3.reshape(1, -1), dinv, Wc, bc.reshape(1, -1))
    return (out[:N], h3[:N])


# speedup vs baseline: 33.8438x; 1.2252x over previous
import functools

import jax
import jax.numpy as jnp
from jax import lax
from jax.experimental import pallas as pl
from jax.experimental.pallas import tpu as pltpu
from jax.experimental.pallas import tpu_sc as plsc

N = 100000
E = 1600000
F_IN = 34

NC = 2
NS = 16
NW = NC * NS
CHUNK = 128
NB = 8
EPW = 50176
EPAD = EPW * NW
NCHUNKS = EPW // CHUNK
NSTEPS = NCHUNKS // NB
NPAD = 102400
RPS = NPAD // NS

_MESH = plsc.VectorSubcoreMesh(
    core_axis_name="c", subcore_axis_name="s", num_cores=NC, num_subcores=NS
)
_SC_PARAMS = pltpu.CompilerParams(use_tc_tiling_on_sc=False)


@functools.partial(
    pl.kernel,
    out_type=jax.ShapeDtypeStruct((NC, NPAD, 16), jnp.float32),
    mesh=_MESH,
    compiler_params=_SC_PARAMS,
    scratch_types=[
        pltpu.VMEM((NB, CHUNK), jnp.int32),
        pltpu.VMEM((CHUNK, 16), jnp.float32),
        pltpu.VMEM_SHARED((NPAD, 16), jnp.float32),
        [pltpu.SemaphoreType.DMA] * NB,
    ],
)
def _deg_kernel(dst_hbm, out_hbm, dst_b, ones_v, acc_sh, ssems):
    c = lax.axis_index("c")
    s = lax.axis_index("s")
    wid = s * NC + c
    r0 = s * RPS

    def _fill_ones(i, carry):
        ones_v[i, :] = jnp.full((16,), 1.0, jnp.float32)
        return carry

    lax.fori_loop(0, CHUNK, _fill_ones, 0)

    def _init(r, carry):
        pltpu.sync_copy(ones_v, acc_sh.at[pl.ds(r0 + r * CHUNK, CHUNK)])
        return carry

    lax.fori_loop(0, RPS // CHUNK, _init, 0)
    plsc.subcore_barrier()

    row0 = wid * NCHUNKS

    def _step(st, carry):
        pltpu.sync_copy(dst_hbm.at[pl.ds(row0 + st * NB, NB)], dst_b)
        descs = [
            pltpu.async_copy(ones_v, acc_sh.at[dst_b.at[b]], ssems[b], add=True)
            for b in range(NB)
        ]
        for d in descs:
            d.wait()
        return carry

    lax.fori_loop(0, NSTEPS, _step, 0)

    plsc.subcore_barrier()
    pltpu.sync_copy(acc_sh.at[pl.ds(r0, RPS)], out_hbm.at[c, pl.ds(r0, RPS)])


def _make_edge_kernel(f):
    @functools.partial(
        pl.kernel,
        out_type=jax.ShapeDtypeStruct((NC, NPAD, f), jnp.float32),
        mesh=_MESH,
        compiler_params=_SC_PARAMS,
        scratch_types=[
            pltpu.VMEM((NB, CHUNK), jnp.int32),
            pltpu.VMEM((NB, CHUNK), jnp.int32),
            pltpu.VMEM((NB, CHUNK, f), jnp.float32),
            pltpu.VMEM_SHARED((NPAD, f), jnp.float32),
            [pltpu.SemaphoreType.DMA] * NB,
            [pltpu.SemaphoreType.DMA] * NB,
        ],
    )
    def _edge_kernel(
        y_hbm, src_hbm, dst_hbm, out_hbm, src_b, dst_b, rows_b, acc_sh, gsems, ssems
    ):
        c = lax.axis_index("c")
        s = lax.axis_index("s")
        wid = s * NC + c
        r0 = s * RPS

        pltpu.sync_copy(y_hbm.at[pl.ds(r0, RPS)], acc_sh.at[pl.ds(r0, RPS)])
        plsc.subcore_barrier()

        row0 = wid * NCHUNKS

        def _step(st, carry):
            pltpu.sync_copy(src_hbm.at[pl.ds(row0 + st * NB, NB)], src_b)
            pltpu.sync_copy(dst_hbm.at[pl.ds(row0 + st * NB, NB)], dst_b)
            gds = [
                pltpu.async_copy(y_hbm.at[src_b.at[b]], rows_b.at[b], gsems[b])
                for b in range(NB)
            ]
            sds = []
            for b in range(NB):
                gds[b].wait()
                sds.append(
                    pltpu.async_copy(
                        rows_b.at[b], acc_sh.at[dst_b.at[b]], ssems[b], add=True
                    )
                )
            for d in sds:
                d.wait()
            return carry

        lax.fori_loop(0, NSTEPS, _step, 0)

        plsc.subcore_barrier()
        pltpu.sync_copy(
            acc_sh.at[pl.ds(r0, RPS)], out_hbm.at[c, pl.ds(r0, RPS)]
        )

    return _edge_kernel


_edge16 = _make_edge_kernel(16)


BR = 2048
GRID = NPAD // BR


def _row_spec(f):
    return pl.BlockSpec((BR, f), lambda i: (i, 0))


def _full_spec(r, f):
    return pl.BlockSpec((r, f), lambda i: (0, 0))


def _part_specs():
    return (
        pl.BlockSpec((1, BR, 16), lambda i: (0, i, 0)),
        pl.BlockSpec((1, BR, 16), lambda i: (1, i, 0)),
    )


def _t1_body(d0_ref, d1_ref, h_ref, w_ref, dinv_ref, y_ref):
    dinv = lax.rsqrt(d0_ref[0, :, 0:1] + d1_ref[0, :, 0:1] - 1.0)
    dinv_ref[...] = dinv
    y_ref[...] = (
        jnp.dot(h_ref[...], w_ref[...], preferred_element_type=jnp.float32) * dinv
    )


def _t1(deg, hpad, W1):
    s0, s1 = _part_specs()
    return pl.pallas_call(
        _t1_body,
        grid=(GRID,),
        in_specs=[s0, s1, _row_spec(F_IN), _full_spec(F_IN, 16)],
        out_specs=[_row_spec(1), _row_spec(16)],
        out_shape=[
            jax.ShapeDtypeStruct((NPAD, 1), jnp.float32),
            jax.ShapeDtypeStruct((NPAD, 16), jnp.float32),
        ],
    )(deg, deg, hpad, W1)


def _make_tmid_body(fi, fo):
    def _tmid_body(y_ref, p0_ref, p1_ref, b_ref, dinv_ref, w_ref, ynext_ref):
        agg = p0_ref[0, :, :fi] + p1_ref[0, :, :fi] - y_ref[:, :fi]
        hn = jnp.tanh(dinv_ref[...] * agg + b_ref[...])
        yn = jnp.dot(hn, w_ref[...], preferred_element_type=jnp.float32) * dinv_ref[...]
        ynext_ref[...] = jnp.concatenate(
            [yn, jnp.zeros((yn.shape[0], 16 - fo), jnp.float32)], axis=1
        )

    return _tmid_body


def _tmid(y, p, b, dinv, W, fi, fo):
    s0, s1 = _part_specs()
    return pl.pallas_call(
        _make_tmid_body(fi, fo),
        grid=(GRID,),
        in_specs=[
            _row_spec(16),
            s0,
            s1,
            _full_spec(1, fi),
            _row_spec(1),
            _full_spec(fi, fo),
        ],
        out_specs=_row_spec(16),
        out_shape=jax.ShapeDtypeStruct((NPAD, 16), jnp.float32),
    )(y, p, p, b, dinv, W)


BRN = 2000
GRID_N = N // BRN


def _t4_body(y_ref, p0_ref, p1_ref, b_ref, dinv_ref, wc_ref, bc_ref, out_ref, h3_ref):
    h3 = jnp.tanh(
        dinv_ref[...] * (p0_ref[0, :, :2] + p1_ref[0, :, :2] - y_ref[:, :2])
        + b_ref[...]
    )
    h3_ref[...] = h3
    out_ref[...] = (
        jnp.dot(h3, wc_ref[...], preferred_element_type=jnp.float32) + bc_ref[...]
    )


def _t4(y, p, b, dinv, Wc, bc):
    return pl.pallas_call(
        _t4_body,
        grid=(GRID_N,),
        in_specs=[
            pl.BlockSpec((BRN, 16), lambda i: (i, 0)),
            pl.BlockSpec((1, BRN, 16), lambda i: (0, i, 0)),
            pl.BlockSpec((1, BRN, 16), lambda i: (1, i, 0)),
            _full_spec(1, 2),
            pl.BlockSpec((BRN, 1), lambda i: (i, 0)),
            _full_spec(2, 4),
            _full_spec(1, 4),
        ],
        out_specs=[
            pl.BlockSpec((BRN, 4), lambda i: (i, 0)),
            pl.BlockSpec((BRN, 2), lambda i: (i, 0)),
        ],
        out_shape=[
            jax.ShapeDtypeStruct((N, 4), jnp.float32),
            jax.ShapeDtypeStruct((N, 2), jnp.float32),
        ],
    )(y, p, p, b, dinv, Wc, bc)


def kernel(h, edge_index, W1, b1, W2, b2, W3, b3, Wc, bc):
    src = edge_index[0]
    dst = edge_index[1]
    pad_idx = jnp.full((EPAD - E,), N, dtype=jnp.int32)
    src_p = jnp.concatenate([src, pad_idx]).reshape(EPAD // CHUNK, CHUNK)
    dst_p = jnp.concatenate([dst, pad_idx]).reshape(EPAD // CHUNK, CHUNK)
    hpad = jnp.concatenate([h, jnp.zeros((NPAD - N, F_IN), h.dtype)])

    deg = _deg_kernel(dst_p)
    dinv, y1 = _t1(deg, hpad, W1)
    p1 = _edge16(y1, src_p, dst_p)
    y2 = _tmid(y1, p1, b1.reshape(1, -1), dinv, W2, 16, 4)
    p2 = _edge16(y2, src_p, dst_p)
    y3 = _tmid(y2, p2, b2.reshape(1, -1), dinv, W3, 4, 2)
    p3 = _edge16(y3, src_p, dst_p)
    out, h3 = _t4(y3, p3, b3.reshape(1, -1), dinv, Wc, bc.reshape(1, -1))
    return (out, h3)
